# Initial kernel scaffold; baseline (speedup 1.0000x reference)
import jax
import jax.numpy as jnp
from jax.experimental import pallas as pl


def kernel(x, weight):
    raise NotImplementedError("write your pallas kernel here")


# speedup vs baseline: 1.6847x; 1.6847x over previous
import functools

import jax
import jax.numpy as jnp
from jax import lax
from jax.experimental import pallas as pl
from jax.experimental.pallas import tpu as pltpu
from jax.experimental.pallas import tpu_sc as plsc

NUM_EMB = 1_000_000
D = 64
BATCH = 16384
HIST = 50
B_TOTAL = BATCH * HIST
NC = 2
NS = 16
NW = NC * NS
PER_W = B_TOTAL // NW
CHUNK = 128
NCHUNK = PER_W // CHUNK

_mesh = plsc.VectorSubcoreMesh(core_axis_name="c", subcore_axis_name="s")


@functools.partial(
    pl.kernel,
    mesh=_mesh,
    compiler_params=pltpu.CompilerParams(use_tc_tiling_on_sc=False),
    out_type=jax.ShapeDtypeStruct((NW * NCHUNK, CHUNK, D), jnp.float32),
    scratch_types=[
        pltpu.VMEM((NCHUNK, CHUNK), jnp.int32),
        pltpu.VMEM((CHUNK, D), jnp.float32),
        pltpu.SemaphoreType.DMA,
    ],
)
def _embed_lookup(idx_hbm, table_hbm, out_hbm, idx_v, rows_v, sem):
    wid = lax.axis_index("s") * NC + lax.axis_index("c")
    pltpu.sync_copy(idx_hbm.at[wid], idx_v)

    def chunk_body(j, carry):
        pltpu.async_copy(table_hbm.at[idx_v.at[j]], rows_v, sem).wait()
        pltpu.sync_copy(rows_v, out_hbm.at[wid * NCHUNK + j])
        return carry

    lax.fori_loop(0, NCHUNK, chunk_body, 0)


def kernel(x, weight):
    idx = x.reshape(NW, NCHUNK, CHUNK).astype(jnp.int32)
    out = _embed_lookup(idx, weight)
    return out.reshape(BATCH, HIST, D)


# speedup vs baseline: 1.8761x; 1.1136x over previous
import functools

import jax
import jax.numpy as jnp
from jax import lax
from jax.experimental import pallas as pl
from jax.experimental.pallas import tpu as pltpu
from jax.experimental.pallas import tpu_sc as plsc

NUM_EMB = 1_000_000
D = 64
BATCH = 16384
HIST = 50
B_TOTAL = BATCH * HIST
NC = 2
NS = 16
NW = NC * NS
PER_W = B_TOTAL // NW
CHUNK = 128
NCHUNK = PER_W // CHUNK
NBUF = 4
NGRP = NCHUNK // NBUF

_mesh = plsc.VectorSubcoreMesh(core_axis_name="c", subcore_axis_name="s")


@functools.partial(
    pl.kernel,
    mesh=_mesh,
    compiler_params=pltpu.CompilerParams(use_tc_tiling_on_sc=False),
    out_type=jax.ShapeDtypeStruct((NW * NCHUNK, CHUNK, D), jnp.float32),
    scratch_types=[
        pltpu.VMEM((NCHUNK, CHUNK), jnp.int32),
        pltpu.VMEM((NBUF, CHUNK, D), jnp.float32),
    ]
    + [pltpu.SemaphoreType.DMA] * (2 * NBUF),
)
def _embed_lookup(idx_hbm, table_hbm, out_hbm, idx_v, rows_v, *sems):
    gsems, ssems = sems[:NBUF], sems[NBUF:]
    wid = lax.axis_index("s") * NC + lax.axis_index("c")
    cbase = wid * NCHUNK
    pltpu.sync_copy(idx_hbm.at[wid], idx_v)

    for b in range(NBUF):
        pltpu.async_copy(table_hbm.at[idx_v.at[b]], rows_v.at[b], gsems[b])

    def group(g, carry):
        for b in range(NBUF):
            j = g * NBUF + b
            pltpu.make_async_copy(
                table_hbm.at[idx_v.at[j]], rows_v.at[b], gsems[b]
            ).wait()
            pltpu.async_copy(rows_v.at[b], out_hbm.at[cbase + j], ssems[b])

            @pl.when(j + NBUF < NCHUNK)
            def _():
                pltpu.make_async_copy(
                    rows_v.at[b], out_hbm.at[cbase + j], ssems[b]
                ).wait()
                pltpu.async_copy(
                    table_hbm.at[idx_v.at[j + NBUF]], rows_v.at[b], gsems[b]
                )

        return carry

    lax.fori_loop(0, NGRP, group, 0)

    for b in range(NBUF):
        pltpu.make_async_copy(rows_v.at[b], out_hbm.at[cbase], ssems[b]).wait()


def kernel(x, weight):
    idx = x.reshape(NW, NCHUNK, CHUNK).astype(jnp.int32)
    out = _embed_lookup(idx, weight)
    return out.reshape(BATCH, HIST, D)
